# baseline (speedup 1.0000x reference)
import jax
import jax.numpy as jnp
from jax.experimental import pallas as pl
from jax.experimental.pallas import tpu as pltpu

S = 2048
D = 4096
E = 64
C = 64
BS = 256
GRID = S // BS


def _gate_block(x_ref, w_ref, cw_ref, dm_ref, carry_ref):
    i = pl.program_id(0)

    @pl.when(i == 0)
    def _():
        carry_ref[...] = jnp.zeros_like(carry_ref)

    x = x_ref[...]
    w = w_ref[...]
    logits = jax.lax.dot_general(
        x, w, (((1,), (1,)), ((), ())),
        preferred_element_type=jnp.float32)

    mx = jnp.max(logits, axis=1, keepdims=True)
    denom = jnp.sum(jnp.exp(logits - mx), axis=1, keepdims=True)
    gate = 1.0 / denom
    eidx = jnp.argmax(logits, axis=1).astype(jnp.int32).reshape(BS, 1)

    ecol = jax.lax.broadcasted_iota(jnp.int32, (BS, E), 1)
    mask = (ecol == eidx).astype(jnp.float32)

    r = jax.lax.broadcasted_iota(jnp.int32, (BS, BS), 0)
    c = jax.lax.broadcasted_iota(jnp.int32, (BS, BS), 1)
    tri = (r >= c).astype(jnp.float32)
    cnt = jax.lax.dot_general(
        tri, mask, (((1,), (0,)), ((), ())),
        preferred_element_type=jnp.float32)

    carry = carry_ref[...]
    locv = cnt - 1.0 + carry
    loc = jnp.sum(locv * mask, axis=1, keepdims=True)
    keep = loc < float(C)
    loci = loc.astype(jnp.int32)
    gk = jnp.where(keep, gate, 0.0)

    carry_ref[...] = carry + jnp.sum(mask, axis=0, keepdims=True)

    e3 = jax.lax.broadcasted_iota(jnp.int32, (BS, E, C), 1)
    c3 = jax.lax.broadcasted_iota(jnp.int32, (BS, E, C), 2)
    hit = (e3 == eidx.reshape(BS, 1, 1)) & (c3 == loci.reshape(BS, 1, 1))
    cw_ref[...] = jnp.where(hit, gk.reshape(BS, 1, 1), 0.0)
    dm_ref[...] = hit & keep.reshape(BS, 1, 1)


def kernel(inp, W):
    x = inp.reshape(-1, inp.shape[-1])
    cw2d, dm2d = pl.pallas_call(
        _gate_block,
        grid=(GRID,),
        in_specs=[
            pl.BlockSpec((BS, D), lambda i: (i, 0)),
            pl.BlockSpec((E, D), lambda i: (0, 0)),
        ],
        out_specs=[
            pl.BlockSpec((BS, E, C), lambda i: (i, 0, 0)),
            pl.BlockSpec((BS, E, C), lambda i: (i, 0, 0)),
        ],
        out_shape=[
            jax.ShapeDtypeStruct((S, E, C), jnp.float32),
            jax.ShapeDtypeStruct((S, E, C), jnp.bool_),
        ],
        scratch_shapes=[pltpu.VMEM((1, E), jnp.float32)],
    )(x, W)
    return cw2d, dm2d


# speedup vs baseline: 3.7352x; 3.7352x over previous
import jax
import jax.numpy as jnp
from jax.experimental import pallas as pl
from jax.experimental.pallas import tpu as pltpu

S = 2048
D = 4096
E = 64
C = 64
BS = 256
GRID = S // BS


def _gate_block(x_ref, w_ref, cw_ref, dm_ref, carry_ref):
    i = pl.program_id(0)

    @pl.when(i == 0)
    def _():
        carry_ref[...] = jnp.zeros_like(carry_ref)

    x = x_ref[...]
    w = w_ref[...]
    lt = jax.lax.dot_general(
        w, x, (((1,), (1,)), ((), ())),
        preferred_element_type=jnp.float32)

    mx = jnp.max(lt, axis=0, keepdims=True)
    denom = jnp.sum(jnp.exp(lt - mx), axis=0, keepdims=True)
    gate = 1.0 / denom

    ismax = (lt == mx).astype(jnp.float32)
    er = jax.lax.broadcasted_iota(jnp.int32, (E, E), 0)
    ec = jax.lax.broadcasted_iota(jnp.int32, (E, E), 1)
    tri_e = (ec <= er).astype(jnp.float32)
    cummax = jax.lax.dot_general(
        tri_e, ismax, (((1,), (0,)), ((), ())),
        preferred_element_type=jnp.float32)
    mask = ismax * (cummax == 1.0)

    tr = jax.lax.broadcasted_iota(jnp.int32, (BS, BS), 0)
    tc = jax.lax.broadcasted_iota(jnp.int32, (BS, BS), 1)
    tri_s = (tr <= tc).astype(jnp.float32)
    cnt = jax.lax.dot_general(
        mask, tri_s, (((1,), (0,)), ((), ())),
        preferred_element_type=jnp.float32)

    carry = carry_ref[...]
    locv = cnt - 1.0 + carry
    loc = jnp.sum(locv * mask, axis=0, keepdims=True)
    loci = loc.astype(jnp.int32)

    carry_ref[...] = carry + jnp.sum(mask, axis=1, keepdims=True)

    cs = jax.lax.broadcasted_iota(jnp.int32, (C, BS), 0)
    ceq = (cs == loci).reshape(1, C, BS)
    hit = (mask != 0.0).reshape(E, 1, BS) & ceq
    cw_ref[...] = jnp.where(hit, gate.reshape(1, 1, BS), 0.0)
    dm_ref[...] = hit.astype(jnp.int8)


def kernel(inp, W):
    x = inp.reshape(-1, inp.shape[-1])
    cw_t, dm_t = pl.pallas_call(
        _gate_block,
        grid=(GRID,),
        in_specs=[
            pl.BlockSpec((BS, D), lambda i: (i, 0)),
            pl.BlockSpec((E, D), lambda i: (0, 0)),
        ],
        out_specs=[
            pl.BlockSpec((E, C, BS), lambda i: (0, 0, i)),
            pl.BlockSpec((E, C, BS), lambda i: (0, 0, i)),
        ],
        out_shape=[
            jax.ShapeDtypeStruct((E, C, S), jnp.float32),
            jax.ShapeDtypeStruct((E, C, S), jnp.int8),
        ],
        scratch_shapes=[pltpu.VMEM((E, 1), jnp.float32)],
    )(x, W)
    cw = jnp.transpose(cw_t, (2, 0, 1))
    dm = jnp.transpose(dm_t, (2, 0, 1)) != 0
    return cw, dm


# speedup vs baseline: 4.1734x; 1.1173x over previous
import jax
import jax.numpy as jnp
from jax.experimental import pallas as pl
from jax.experimental.pallas import tpu as pltpu

S = 2048
D = 4096
E = 64
C = 64
BS = 256
GRID = S // BS


def _gate_block(x_ref, w_ref, cw_ref, code_ref, carry_ref):
    i = pl.program_id(0)

    @pl.when(i == 0)
    def _():
        carry_ref[...] = jnp.zeros_like(carry_ref)

    x = x_ref[...]
    w = w_ref[...]
    lt = jax.lax.dot_general(
        w, x, (((1,), (1,)), ((), ())),
        preferred_element_type=jnp.float32)

    mx = jnp.max(lt, axis=0, keepdims=True)
    denom = jnp.sum(jnp.exp(lt - mx), axis=0, keepdims=True)
    gate = 1.0 / denom

    ismax = (lt == mx).astype(jnp.float32)
    er = jax.lax.broadcasted_iota(jnp.int32, (E, E), 0)
    ec = jax.lax.broadcasted_iota(jnp.int32, (E, E), 1)
    tri_e = (ec <= er).astype(jnp.float32)
    cummax = jax.lax.dot_general(
        tri_e, ismax, (((1,), (0,)), ((), ())),
        preferred_element_type=jnp.float32)
    mask = ismax * (cummax == 1.0)

    tr = jax.lax.broadcasted_iota(jnp.int32, (BS, BS), 0)
    tc = jax.lax.broadcasted_iota(jnp.int32, (BS, BS), 1)
    tri_s = (tr <= tc).astype(jnp.float32)
    cnt = jax.lax.dot_general(
        mask, tri_s, (((1,), (0,)), ((), ())),
        preferred_element_type=jnp.float32)

    carry = carry_ref[...]
    locv = cnt - 1.0 + carry
    loc = jnp.sum(locv * mask, axis=0, keepdims=True)
    loci = loc.astype(jnp.int32)

    carry_ref[...] = carry + jnp.sum(mask, axis=1, keepdims=True)

    cs = jax.lax.broadcasted_iota(jnp.int32, (C, BS), 0)
    ceq = (cs == loci).reshape(1, C, BS)
    hit = (mask != 0.0).reshape(E, 1, BS) & ceq
    cw_ref[...] = jnp.where(hit, gate.reshape(1, 1, BS), 0.0)

    ei = jax.lax.broadcasted_iota(jnp.int32, (E, BS), 0)
    eidx = jnp.sum(ei * mask.astype(jnp.int32), axis=0, keepdims=True)
    code_ref[...] = (eidx << 12) | loci


def kernel(inp, W):
    x = inp.reshape(-1, inp.shape[-1])
    cw_t, code = pl.pallas_call(
        _gate_block,
        grid=(GRID,),
        in_specs=[
            pl.BlockSpec((BS, D), lambda i: (i, 0)),
            pl.BlockSpec((E, D), lambda i: (0, 0)),
        ],
        out_specs=[
            pl.BlockSpec((E, C, BS), lambda i: (0, 0, i)),
            pl.BlockSpec((1, BS), lambda i: (0, i)),
        ],
        out_shape=[
            jax.ShapeDtypeStruct((E, C, S), jnp.float32),
            jax.ShapeDtypeStruct((1, S), jnp.int32),
        ],
        scratch_shapes=[pltpu.VMEM((E, 1), jnp.float32)],
    )(x, W)
    cw = jnp.transpose(cw_t, (2, 0, 1))
    c0 = code.reshape(S, 1, 1)
    e_i = jax.lax.broadcasted_iota(jnp.int32, (S, E, C), 1)
    c_i = jax.lax.broadcasted_iota(jnp.int32, (S, E, C), 2)
    dm = (e_i == (c0 >> 12)) & (c_i == (c0 & 4095))
    return cw, dm


# speedup vs baseline: 4.2712x; 1.0234x over previous
import jax
import jax.numpy as jnp
from jax.experimental import pallas as pl
from jax.experimental.pallas import tpu as pltpu

S = 2048
D = 4096
E = 64
C = 64
BS = 512
GRID = S // BS


def _gate_block(x_ref, w_ref, cw_ref, code_ref, carry_ref):
    i = pl.program_id(0)

    @pl.when(i == 0)
    def _():
        carry_ref[...] = jnp.zeros_like(carry_ref)

    x = x_ref[...]
    w = w_ref[...]
    lt = jax.lax.dot_general(
        w, x, (((1,), (1,)), ((), ())),
        preferred_element_type=jnp.float32)

    mx = jnp.max(lt, axis=0, keepdims=True)
    denom = jnp.sum(jnp.exp(lt - mx), axis=0, keepdims=True)
    gate = 1.0 / denom

    ismax = (lt == mx).astype(jnp.float32)
    er = jax.lax.broadcasted_iota(jnp.int32, (E, E), 0)
    ec = jax.lax.broadcasted_iota(jnp.int32, (E, E), 1)
    tri_e = (ec <= er).astype(jnp.float32)
    cummax = jax.lax.dot_general(
        tri_e, ismax, (((1,), (0,)), ((), ())),
        preferred_element_type=jnp.float32)
    mask = ismax * (cummax == 1.0)

    tr = jax.lax.broadcasted_iota(jnp.int32, (BS, BS), 0)
    tc = jax.lax.broadcasted_iota(jnp.int32, (BS, BS), 1)
    tri_s = (tr <= tc).astype(jnp.float32)
    cnt = jax.lax.dot_general(
        mask, tri_s, (((1,), (0,)), ((), ())),
        preferred_element_type=jnp.float32)

    carry = carry_ref[...]
    locv = cnt - 1.0 + carry
    loc = jnp.sum(locv * mask, axis=0, keepdims=True)
    loci = loc.astype(jnp.int32)

    carry_ref[...] = carry + jnp.sum(mask, axis=1, keepdims=True)

    cs = jax.lax.broadcasted_iota(jnp.int32, (C, BS), 0)
    ceq = (cs == loci).reshape(1, C, BS)
    hit = (mask != 0.0).reshape(E, 1, BS) & ceq
    cw_ref[...] = jnp.where(hit, gate.reshape(1, 1, BS), 0.0)

    ei = jax.lax.broadcasted_iota(jnp.int32, (E, BS), 0)
    eidx = jnp.sum(ei * mask.astype(jnp.int32), axis=0, keepdims=True)
    code_ref[...] = (eidx << 12) | loci


def kernel(inp, W):
    x = inp.reshape(-1, inp.shape[-1])
    cw_t, code = pl.pallas_call(
        _gate_block,
        grid=(GRID,),
        in_specs=[
            pl.BlockSpec((BS, D), lambda i: (i, 0)),
            pl.BlockSpec((E, D), lambda i: (0, 0)),
        ],
        out_specs=[
            pl.BlockSpec((E, C, BS), lambda i: (0, 0, i)),
            pl.BlockSpec((1, BS), lambda i: (0, i)),
        ],
        out_shape=[
            jax.ShapeDtypeStruct((E, C, S), jnp.float32),
            jax.ShapeDtypeStruct((1, S), jnp.int32),
        ],
        scratch_shapes=[pltpu.VMEM((E, 1), jnp.float32)],
    )(x, W)
    cw = jnp.transpose(cw_t, (2, 0, 1))
    c0 = code.reshape(S, 1, 1)
    e_i = jax.lax.broadcasted_iota(jnp.int32, (S, E, C), 1)
    c_i = jax.lax.broadcasted_iota(jnp.int32, (S, E, C), 2)
    dm = (e_i == (c0 >> 12)) & (c_i == (c0 & 4095))
    return cw, dm
